# baseline (speedup 1.0000x reference)
import functools

import jax
import jax.numpy as jnp
from jax import lax
from jax.experimental import pallas as pl
from jax.experimental.pallas import tpu as pltpu
from jax.experimental.pallas import tpu_sc as plsc

_NC = 2
_NS = 16
_NW = _NC * _NS


@functools.lru_cache(maxsize=None)
def _make_gather(B, V, D, C):
  b_per_w = B // _NW
  n_chunks = b_per_w // C
  mesh = plsc.VectorSubcoreMesh(core_axis_name="c", subcore_axis_name="s")

  @functools.partial(
      pl.kernel,
      mesh=mesh,
      out_type=jax.ShapeDtypeStruct((B, D), jnp.float32),
      scratch_types=[
          pltpu.VMEM((C,), jnp.int32),
          pltpu.VMEM((C, D), jnp.float32),
          pltpu.SemaphoreType.DMA,
      ],
      compiler_params=pltpu.CompilerParams(use_tc_tiling_on_sc=False),
  )
  def k(idx_hbm, table_hbm, out_hbm, idx_v, rows_v, sem):
    wid = lax.axis_index("s") * _NC + lax.axis_index("c")
    base = wid * b_per_w

    def body(i, carry):
      off = base + i * C
      pltpu.sync_copy(idx_hbm.at[pl.ds(off, C)], idx_v)
      pltpu.async_copy(table_hbm.at[idx_v], rows_v, sem).wait()
      pltpu.sync_copy(rows_v, out_hbm.at[pl.ds(off, C)])
      return carry

    lax.fori_loop(0, n_chunks, body, 0)

  return k


def kernel(indices, table):
  R, S = indices.shape
  V, D = table.shape
  B = R * S
  idx_flat = indices.reshape(B).astype(jnp.int32)
  out = _make_gather(B, V, D, 1600)(idx_flat, table)
  return out.reshape(R, S, D)


# speedup vs baseline: 1.0011x; 1.0011x over previous
import functools

import jax
import jax.numpy as jnp
from jax import lax
from jax.experimental import pallas as pl
from jax.experimental.pallas import tpu as pltpu
from jax.experimental.pallas import tpu_sc as plsc

_NC = 2
_NS = 16
_NW = _NC * _NS
_NBUF = 2


@functools.lru_cache(maxsize=None)
def _make_gather(B, V, D, C):
  b_per_w = B // _NW
  n_chunks = b_per_w // C
  mesh = plsc.VectorSubcoreMesh(core_axis_name="c", subcore_axis_name="s")

  @functools.partial(
      pl.kernel,
      mesh=mesh,
      out_type=jax.ShapeDtypeStruct((B, D), jnp.float32),
      scratch_types=[
          pltpu.VMEM((n_chunks, C), jnp.int32),
          [pltpu.VMEM((C, D), jnp.float32) for _ in range(_NBUF)],
          [pltpu.SemaphoreType.DMA for _ in range(_NBUF)],
          [pltpu.SemaphoreType.DMA for _ in range(_NBUF)],
      ],
      compiler_params=pltpu.CompilerParams(use_tc_tiling_on_sc=False),
  )
  def k(idx_hbm, table_hbm, out_hbm, idx_v, bufs, gsems, wsems):
    wid = lax.axis_index("s") * _NC + lax.axis_index("c")
    base = wid * b_per_w
    pltpu.sync_copy(idx_hbm.at[wid], idx_v)

    gathers = [None] * n_chunks
    writes = [None] * n_chunks

    def start_gather(j):
      b = j % _NBUF
      gathers[j] = pltpu.async_copy(table_hbm.at[idx_v.at[j]], bufs[b],
                                    gsems[b])

    start_gather(0)
    for j in range(n_chunks):
      b = j % _NBUF
      gathers[j].wait()
      if j + 1 < n_chunks:
        if j + 1 >= _NBUF:
          writes[j + 1 - _NBUF].wait()
        start_gather(j + 1)
      writes[j] = pltpu.async_copy(
          bufs[b], out_hbm.at[pl.ds(base + j * C, C)], wsems[b])
    for j in range(max(0, n_chunks - _NBUF), n_chunks):
      writes[j].wait()

  return k


def kernel(indices, table):
  R, S = indices.shape
  V, D = table.shape
  B = R * S
  C = 1600
  n_chunks = (B // _NW) // C
  idx = indices.reshape(_NW, n_chunks, C).astype(jnp.int32)
  out = _make_gather(B, V, D, C)(idx, table)
  return out.reshape(R, S, D)
